# Initial kernel scaffold; baseline (speedup 1.0000x reference)
import jax
import jax.numpy as jnp
from jax.experimental import pallas as pl


def kernel(x, mean, std, mem_features, mem_labels):
    raise NotImplementedError("write your pallas kernel here")


# speedup vs baseline: 2.4519x; 2.4519x over previous
import functools

import jax
import jax.numpy as jnp
from jax import lax
from jax.experimental import pallas as pl
from jax.experimental.pallas import tpu as pltpu
from jax.experimental.pallas import tpu_sc as plsc

K = 16
TAU = 0.2
SCALE = 20.0
NCLS = 1000
B = 1024
D = 256
N = 100000
CT = 2048
NPAD = 100352
NSTEP = NPAD // CT
NB = NPAD // 128
NW = 32
RPW = B // NW
MAXC = 4096
FSZ = 832
FTRASH = FSZ - 1
NEG = -1e30


def _mm_body(x_ref, mean_ref, std_ref, mf_ref, sims_ref, bm_ref, xn_ref):
    j = pl.program_id(0)

    @pl.when(j == 0)
    def _():
        xx = x_ref[...]
        xx = (xx - mean_ref[...]) / std_ref[...]
        nrm = jnp.sqrt(jnp.sum(xx * xx, axis=1, keepdims=True))
        xn_ref[...] = xx / jnp.maximum(nrm, 1e-6)

    xn = xn_ref[...]
    s = lax.dot_general(xn, mf_ref[...], (((1,), (1,)), ((), ())),
                        preferred_element_type=jnp.float32)
    col = j * CT + lax.broadcasted_iota(jnp.int32, (B, CT), 1)
    s = jnp.where(col < N, s, NEG)
    sims_ref[...] = s
    bm_ref[...] = jnp.max(s.reshape(B, CT // 128, 128), axis=-1).reshape(B // 8, 1, 8, CT // 128)


def _phase_a(x, mean, std, mf_pad):
    return pl.pallas_call(
        _mm_body,
        grid=(NSTEP,),
        in_specs=[
            pl.BlockSpec((B, D), lambda j: (0, 0)),
            pl.BlockSpec((1, D), lambda j: (0, 0)),
            pl.BlockSpec((1, D), lambda j: (0, 0)),
            pl.BlockSpec((CT, D), lambda j: (j, 0)),
        ],
        out_specs=[
            pl.BlockSpec((B, CT), lambda j: (0, j)),
            pl.BlockSpec((B // 8, 1, 8, CT // 128), lambda j: (0, j, 0, 0)),
        ],
        out_shape=[
            jax.ShapeDtypeStruct((B, NPAD), jnp.float32),
            jax.ShapeDtypeStruct((B // 8, NSTEP, 8, CT // 128), jnp.float32),
        ],
        scratch_shapes=[pltpu.VMEM((B, D), jnp.float32)],
    )(x, mean.reshape(1, D), std.reshape(1, D), mf_pad)


def _cmpx_desc(iota, v, ti, j):
    p = jnp.bitwise_xor(iota, j)
    pk = v[p]
    pi = ti[p]
    up = (iota & j) == 0
    ge = v >= pk
    le = v <= pk
    nv = jnp.where(up, jnp.maximum(v, pk), jnp.minimum(v, pk))
    ni = jnp.where(up, jnp.where(ge, ti, pi), jnp.where(le, ti, pi))
    return nv, ni


def _sort16_desc(iota, v, ti):
    for m in (1, 2, 4, 8):
        pos = iota & (2 * m - 1)
        perm = jnp.where(pos >= m, iota - pos + (3 * m - 1 - pos), iota)
        v = v[perm]
        ti = ti[perm]
        j = m
        while j >= 1:
            v, ti = _cmpx_desc(iota, v, ti, j)
            j //= 2
    return v, ti


def _merge16_desc(iota, tv, ti, sv, si):
    rl = 15 - iota
    rv = sv[rl]
    ri = si[rl]
    keep = tv >= rv
    nv = jnp.where(keep, tv, rv)
    ni = jnp.where(keep, ti, ri)
    for j in (8, 4, 2, 1):
        nv, ni = _cmpx_desc(iota, nv, ni, j)
    return nv, ni


def _lane_reduce(iota, v, op):
    for sh in (8, 4, 2, 1):
        v = op(v, v[jnp.bitwise_and(iota + sh, 15)])
    return v


def _prefix_count(iota, m):
    s = jnp.where(m, jnp.full((16,), 1, jnp.int32), jnp.full((16,), 0, jnp.int32))
    for sh in (1, 2, 4, 8):
        g = s[jnp.maximum(iota - sh, 0)]
        s = s + jnp.where(iota >= sh, g, 0)
    return s


def _topk_body(sims_ref, bm_ref, lbl8_ref, ow_ref, ol8_ref,
               bmbuf, blkbuf, cval, ccol, wgbuf, fsm, offr, tsm, sem, sem2):
    wid = lax.axis_index("s") * 2 + lax.axis_index("c")
    iota = lax.iota(jnp.int32, 16)

    def grp_body(gi, _):
        r0 = wid * RPW + gi * 8
        r8 = wid * 4 + gi
        pltpu.sync_copy(bm_ref.at[r8], bmbuf)

        gms = [jnp.full((16,), NEG, jnp.float32) for _ in range(8)]
        for j in range(NSTEP):
            for q in range(8):
                gms[q] = jnp.maximum(gms[q], bmbuf[j * 8 + q])
        for q in range(8):
            tsm[q] = _lane_reduce(iota, gms[q], jnp.minimum)[0]

        def row_body(q, _):
            r = r0 + q
            t = tsm[q]

            offr[0] = 0
            offr[1] = 0

            for j in range(NSTEP):
                v = bmbuf[j * 8 + q]
                mxs = _lane_reduce(iota, v, jnp.maximum)[0]

                @pl.when(mxs >= t)
                def _(v=v, j=j):
                    o = offr[0]
                    for g in range(16):
                        fsm[o] = j * 16 + g
                        o = o + jnp.where(v[g] >= t, 1, 0)
                    offr[0] = o
            nb_s = offr[0]
            for g in range(16):
                fsm[nb_s + g] = NB - 1

            nchunks = (nb_s + 15) // 16

            def make_iv(ci):
                base = ci * 16
                iv = jnp.zeros((16,), jnp.int32)
                for g in range(16):
                    iv = jnp.where(iota == g, lax.broadcast(fsm[base + g], (16,)), iv)
                return iv

            pltpu.async_copy(sims_ref.at[make_iv(0) + r * NB],
                             blkbuf.at[pl.ds(0, 16)], sem)

            def chunk(ci, _):
                par16 = jnp.bitwise_and(ci, 1) * 16
                npar16 = 16 - par16

                @pl.when(ci + 1 < nchunks)
                def _():
                    pltpu.async_copy(sims_ref.at[make_iv(ci + 1) + r * NB],
                                     blkbuf.at[pl.ds(npar16, 16)], sem)
                pltpu.make_async_copy(sims_ref.at[pl.ds(0, 16)],
                                      blkbuf.at[pl.ds(par16, 16)], sem).wait()
                base = ci * 16
                o2 = offr[1]
                for g in range(16):
                    cb = fsm[base + g] * 128
                    for s_ in range(8):
                        v = blkbuf[par16 + g, pl.ds(s_ * 16, 16)]
                        mxs = _lane_reduce(iota, v, jnp.maximum)[0]
                        oc = jnp.minimum(o2, MAXC - 16)
                        cval[pl.ds(oc, 16)] = jnp.where(v >= t, v, NEG)
                        ccol[pl.ds(oc, 16)] = cb + s_ * 16 + iota
                        o2 = oc + jnp.where(mxs >= t, 16, 0)
                offr[1] = o2
                return 0
            lax.fori_loop(0, nchunks, chunk, 0)
            nc = offr[1]

            def sel(j, carry):
                tv, ti = carry
                cv = cval[pl.ds(j * 16, 16)]
                ci = ccol[pl.ds(j * 16, 16)]
                sv, si = _sort16_desc(iota, cv, ci)
                return _merge16_desc(iota, tv, ti, sv, si)
            tv, ti = lax.fori_loop(
                0, nc // 16, sel,
                (jnp.full((16,), NEG, jnp.float32), jnp.zeros((16,), jnp.int32)))

            e = jnp.exp((tv - tv[0]) * (1.0 / TAU))
            se = _lane_reduce(iota, e, jnp.add)
            wgbuf[q] = e * SCALE / se
            descs = [pltpu.async_copy(lbl8_ref.at[ti[g]], ol8_ref.at[r, g], sem2)
                     for g in range(16)]
            for dsc in descs:
                dsc.wait()
            return 0

        lax.fori_loop(0, 8, row_body, 0)
        pltpu.sync_copy(wgbuf, ow_ref.at[pl.ds(r0, 8)])
        return 0

    lax.fori_loop(0, RPW // 8, grp_body, 0)


def _phase_b(sims2d, bm2d, labels8):
    mesh = plsc.VectorSubcoreMesh(core_axis_name="c", subcore_axis_name="s")
    kern = functools.partial(
        pl.kernel,
        out_type=[
            jax.ShapeDtypeStruct((B, K), jnp.float32),
            jax.ShapeDtypeStruct((B, K, 8), jnp.int32),
        ],
        mesh=mesh,
        scratch_types=[
            pltpu.VMEM((NSTEP * 8, 16), jnp.float32),
            pltpu.VMEM((32, 128), jnp.float32),
            pltpu.VMEM((MAXC,), jnp.float32),
            pltpu.VMEM((MAXC,), jnp.int32),
            pltpu.VMEM((8, 16), jnp.float32),
            pltpu.SMEM((800,), jnp.int32),
            pltpu.SMEM((4,), jnp.int32),
            pltpu.SMEM((8,), jnp.float32),
            pltpu.SemaphoreType.DMA,
            pltpu.SemaphoreType.DMA,
        ],
    )(_topk_body)
    return kern(sims2d, bm2d, labels8)


def _scatter_body(w_ref, l_ref, o_ref):
    cols = lax.broadcasted_iota(jnp.int32, (B, NCLS), 1)
    acc = jnp.zeros((B, NCLS), jnp.float32)
    for k in range(K):
        acc = acc + w_ref[:, k:k + 1] * (l_ref[:, k:k + 1] == cols).astype(jnp.float32)
    o_ref[...] = acc


def _phase_c(w, l):
    return pl.pallas_call(
        _scatter_body,
        out_shape=jax.ShapeDtypeStruct((B, NCLS), jnp.float32),
    )(w, l)


def kernel(x, mean, std, mem_features, mem_labels):
    mf_pad = jnp.pad(mem_features, ((0, NPAD - N), (0, 0)))
    sims, bm3d = _phase_a(x, mean, std, mf_pad)
    sims2d = sims.reshape(B * NB, 128)
    bm2d = bm3d.reshape(B // 8, NSTEP * 8, 16)
    labels8 = jnp.tile(mem_labels[:, None], (1, 8))
    w, lbl8 = _phase_b(sims2d, bm2d, labels8)
    return _phase_c(w, lbl8[:, :, 0])


# speedup vs baseline: 3.0855x; 1.2584x over previous
import functools

import jax
import jax.numpy as jnp
from jax import lax
from jax.experimental import pallas as pl
from jax.experimental.pallas import tpu as pltpu
from jax.experimental.pallas import tpu_sc as plsc

K = 16
TAU = 0.2
SCALE = 20.0
NCLS = 1000
B = 1024
D = 256
N = 100000
CT = 2048
NPAD = 100352
NSTEP = NPAD // CT
NB = NPAD // 128
NW = 32
RPW = B // NW
MAXC = 4096
FSZ = 832
FTRASH = FSZ - 1
NEG = -1e30


def _mm_body(x_ref, mean_ref, std_ref, mf_ref, sims_ref, bm_ref, xn_ref):
    j = pl.program_id(0)

    @pl.when(j == 0)
    def _():
        xx = x_ref[...]
        xx = (xx - mean_ref[...]) / std_ref[...]
        nrm = jnp.sqrt(jnp.sum(xx * xx, axis=1, keepdims=True))
        xn_ref[...] = xx / jnp.maximum(nrm, 1e-6)

    xn = xn_ref[...]
    s = lax.dot_general(xn, mf_ref[...], (((1,), (1,)), ((), ())),
                        preferred_element_type=jnp.float32)
    col = j * CT + lax.broadcasted_iota(jnp.int32, (B, CT), 1)
    s = jnp.where(col < N, s, NEG)
    sims_ref[...] = s
    bm_ref[...] = jnp.max(s.reshape(B, CT // 128, 128), axis=-1).reshape(B // 8, 1, 8, CT // 128)


def _phase_a(x, mean, std, mf_pad):
    return pl.pallas_call(
        _mm_body,
        grid=(NSTEP,),
        in_specs=[
            pl.BlockSpec((B, D), lambda j: (0, 0)),
            pl.BlockSpec((1, D), lambda j: (0, 0)),
            pl.BlockSpec((1, D), lambda j: (0, 0)),
            pl.BlockSpec((CT, D), lambda j: (j, 0)),
        ],
        out_specs=[
            pl.BlockSpec((B, CT), lambda j: (0, j)),
            pl.BlockSpec((B // 8, 1, 8, CT // 128), lambda j: (0, j, 0, 0)),
        ],
        out_shape=[
            jax.ShapeDtypeStruct((B, NPAD), jnp.float32),
            jax.ShapeDtypeStruct((B // 8, NSTEP, 8, CT // 128), jnp.float32),
        ],
        scratch_shapes=[pltpu.VMEM((B, D), jnp.float32)],
    )(x, mean.reshape(1, D), std.reshape(1, D), mf_pad)


def _cmpx_desc(iota, v, ti, j):
    p = jnp.bitwise_xor(iota, j)
    pk = v[p]
    pi = ti[p]
    up = (iota & j) == 0
    ge = v >= pk
    le = v <= pk
    nv = jnp.where(up, jnp.maximum(v, pk), jnp.minimum(v, pk))
    ni = jnp.where(up, jnp.where(ge, ti, pi), jnp.where(le, ti, pi))
    return nv, ni


def _sort16_desc(iota, v, ti):
    for m in (1, 2, 4, 8):
        pos = iota & (2 * m - 1)
        perm = jnp.where(pos >= m, iota - pos + (3 * m - 1 - pos), iota)
        v = v[perm]
        ti = ti[perm]
        j = m
        while j >= 1:
            v, ti = _cmpx_desc(iota, v, ti, j)
            j //= 2
    return v, ti


def _merge16_desc(iota, tv, ti, sv, si):
    rl = 15 - iota
    rv = sv[rl]
    ri = si[rl]
    keep = tv >= rv
    nv = jnp.where(keep, tv, rv)
    ni = jnp.where(keep, ti, ri)
    for j in (8, 4, 2, 1):
        nv, ni = _cmpx_desc(iota, nv, ni, j)
    return nv, ni


def _lane_reduce(iota, v, op):
    for sh in (8, 4, 2, 1):
        v = op(v, v[jnp.bitwise_and(iota + sh, 15)])
    return v


def _prefix_count(iota, m):
    s = jnp.where(m, jnp.full((16,), 1, jnp.int32), jnp.full((16,), 0, jnp.int32))
    for sh in (1, 2, 4, 8):
        g = s[jnp.maximum(iota - sh, 0)]
        s = s + jnp.where(iota >= sh, g, 0)
    return s


def _topk_body(sims_ref, bm_ref, lbl8_ref, ow_ref, ol8_ref,
               bmbuf, blkbuf, cval, ccol, wgbuf, fsm, offr, tsm, sem, sem2):
    wid = lax.axis_index("s") * 2 + lax.axis_index("c")
    iota = lax.iota(jnp.int32, 16)

    def grp_body(gi, _):
        r0 = wid * RPW + gi * 8
        r8 = wid * 4 + gi
        pltpu.sync_copy(bm_ref.at[r8], bmbuf)

        def mxall(j, accs):
            base = j * 8
            return tuple(jnp.maximum(accs[q], bmbuf[base + q]) for q in range(8))
        gms = lax.fori_loop(
            0, NSTEP, mxall,
            tuple(jnp.full((16,), NEG, jnp.float32) for _ in range(8)))
        for q in range(8):
            tsm[q] = _lane_reduce(iota, gms[q], jnp.minimum)[0]

        def row_body(q, _):
            r = r0 + q
            t = tsm[q]

            offr[0] = 0
            offr[1] = 0

            def compact(j, _):
                v = bmbuf[j * 8 + q]
                mxs = _lane_reduce(iota, v, jnp.maximum)[0]

                @pl.when(mxs >= t)
                def _():
                    o = offr[0]
                    for g in range(16):
                        fsm[o] = j * 16 + g
                        o = o + jnp.where(v[g] >= t, 1, 0)
                    offr[0] = o
                return 0
            lax.fori_loop(0, NSTEP, compact, 0)
            nb_s = offr[0]
            for g in range(16):
                fsm[nb_s + g] = NB - 1

            nchunks = (nb_s + 15) // 16

            def make_iv(ci):
                base = ci * 16
                iv = jnp.zeros((16,), jnp.int32)
                for g in range(16):
                    iv = jnp.where(iota == g, lax.broadcast(fsm[base + g], (16,)), iv)
                return iv

            pltpu.async_copy(sims_ref.at[make_iv(0) + r * NB],
                             blkbuf.at[pl.ds(0, 16)], sem)

            def chunk(ci, _):
                par16 = jnp.bitwise_and(ci, 1) * 16
                npar16 = 16 - par16

                @pl.when(ci + 1 < nchunks)
                def _():
                    pltpu.async_copy(sims_ref.at[make_iv(ci + 1) + r * NB],
                                     blkbuf.at[pl.ds(npar16, 16)], sem)
                pltpu.make_async_copy(sims_ref.at[pl.ds(0, 16)],
                                      blkbuf.at[pl.ds(par16, 16)], sem).wait()
                base = ci * 16
                o2 = offr[1]
                for g in range(16):
                    cb = fsm[base + g] * 128
                    for s_ in range(8):
                        v = blkbuf[par16 + g, pl.ds(s_ * 16, 16)]
                        mxs = _lane_reduce(iota, v, jnp.maximum)[0]
                        oc = jnp.minimum(o2, MAXC - 16)
                        cval[pl.ds(oc, 16)] = jnp.where(v >= t, v, NEG)
                        ccol[pl.ds(oc, 16)] = cb + s_ * 16 + iota
                        o2 = oc + jnp.where(mxs >= t, 16, 0)
                offr[1] = o2
                return 0
            lax.fori_loop(0, nchunks, chunk, 0)
            nc = offr[1]

            def sel(j, carry):
                tv, ti = carry
                cv = cval[pl.ds(j * 16, 16)]
                ci = ccol[pl.ds(j * 16, 16)]
                sv, si = _sort16_desc(iota, cv, ci)
                return _merge16_desc(iota, tv, ti, sv, si)
            tv, ti = lax.fori_loop(
                0, nc // 16, sel,
                (jnp.full((16,), NEG, jnp.float32), jnp.zeros((16,), jnp.int32)))

            e = jnp.exp((tv - tv[0]) * (1.0 / TAU))
            se = _lane_reduce(iota, e, jnp.add)
            wgbuf[q] = e * SCALE / se
            for g in range(16):
                pltpu.async_copy(lbl8_ref.at[ti[g]], ol8_ref.at[r, g], sem2)
            return 0

        lax.fori_loop(0, 8, row_body, 0)
        pltpu.sync_copy(wgbuf, ow_ref.at[pl.ds(r0, 8)])
        return 0

    lax.fori_loop(0, RPW // 8, grp_body, 0)
    pltpu.make_async_copy(ol8_ref.at[pl.ds(0, RPW)],
                          ol8_ref.at[pl.ds(wid * RPW, RPW)], sem2).wait()


def _phase_b(sims2d, bm2d, labels8):
    mesh = plsc.VectorSubcoreMesh(core_axis_name="c", subcore_axis_name="s")
    kern = functools.partial(
        pl.kernel,
        out_type=[
            jax.ShapeDtypeStruct((B, K), jnp.float32),
            jax.ShapeDtypeStruct((B, K, 8), jnp.int32),
        ],
        mesh=mesh,
        scratch_types=[
            pltpu.VMEM((NSTEP * 8, 16), jnp.float32),
            pltpu.VMEM((32, 128), jnp.float32),
            pltpu.VMEM((MAXC,), jnp.float32),
            pltpu.VMEM((MAXC,), jnp.int32),
            pltpu.VMEM((8, 16), jnp.float32),
            pltpu.SMEM((800,), jnp.int32),
            pltpu.SMEM((4,), jnp.int32),
            pltpu.SMEM((8,), jnp.float32),
            pltpu.SemaphoreType.DMA,
            pltpu.SemaphoreType.DMA,
        ],
    )(_topk_body)
    return kern(sims2d, bm2d, labels8)


def _scatter_body(w_ref, l_ref, o_ref):
    cols = lax.broadcasted_iota(jnp.int32, (B, NCLS), 1)
    acc = jnp.zeros((B, NCLS), jnp.float32)
    for k in range(K):
        acc = acc + w_ref[:, k:k + 1] * (l_ref[:, k:k + 1] == cols).astype(jnp.float32)
    o_ref[...] = acc


def _phase_c(w, l):
    return pl.pallas_call(
        _scatter_body,
        out_shape=jax.ShapeDtypeStruct((B, NCLS), jnp.float32),
    )(w, l)


def kernel(x, mean, std, mem_features, mem_labels):
    mf_pad = jnp.pad(mem_features, ((0, NPAD - N), (0, 0)))
    sims, bm3d = _phase_a(x, mean, std, mf_pad)
    sims2d = sims.reshape(B * NB, 128)
    bm2d = bm3d.reshape(B // 8, NSTEP * 8, 16)
    labels8 = jnp.tile(mem_labels[:, None], (1, 8))
    w, lbl8 = _phase_b(sims2d, bm2d, labels8)
    return _phase_c(w, lbl8[:, :, 0])
